# baseline (speedup 1.0000x reference)
import functools

import jax
import jax.numpy as jnp
from jax import lax
from jax.experimental import pallas as pl
from jax.experimental.pallas import tpu as pltpu
from jax.experimental.pallas import tpu_sc as plsc

N_USERS = 25000
N_ITEMS = 25000
N = N_USERS + N_ITEMS
E = 800000
D = 64
H = 32

NC = 2
NS = 16
CH = 128

N_PAD = 50176
STRIPE = N_PAD // NS
E_PAD = 819200
K = E_PAD // CH
NGC2 = K // (NC * NS)
NGC3 = K // NS
RING = 8
VR = 4

_mesh = plsc.VectorSubcoreMesh(core_axis_name="c", subcore_axis_name="s")
_sc_params = pltpu.CompilerParams(use_tc_tiling_on_sc=False)


def _out2(width):
    return [jax.ShapeDtypeStruct((N_PAD, width), jnp.float32),
            jax.ShapeDtypeStruct((N_PAD, width), jnp.float32)]


@functools.partial(
    pl.kernel,
    out_type=_out2(16),
    mesh=_mesh,
    compiler_params=_sc_params,
    scratch_types=[
        pltpu.VMEM((RING, CH), jnp.int32),
        pltpu.VMEM((CH, 16), jnp.float32),
        pltpu.VMEM_SHARED((N_PAD, 16), jnp.float32),
        pltpu.SemaphoreType.DMA,
        pltpu.SemaphoreType.DMA,
    ],
)
def _sc_deg(colk_hbm, ones_hbm, zeros_hbm, out0_hbm, out1_hbm,
            cidx, ones_v, accum, isem, ssem):
    c = lax.axis_index("c")
    s = lax.axis_index("s")
    base = (c * NS + s) * NGC2
    for j in range(4):
        pltpu.async_copy(colk_hbm.at[base + j], cidx.at[j], isem)
    pltpu.sync_copy(ones_hbm, ones_v)
    pltpu.sync_copy(zeros_hbm, accum.at[pl.ds(s * STRIPE, STRIPE)])
    plsc.subcore_barrier()

    @pl.loop(0, NGC2 // RING)
    def _(u):
        for j in range(RING):
            g = u * RING + j

            @pl.when(g >= 2)
            def _():
                pltpu.make_async_copy(ones_v, accum.at[cidx.at[j]], ssem).wait()

            @pl.when(g + 4 < NGC2)
            def _():
                pltpu.async_copy(colk_hbm.at[base + g + 4],
                                 cidx.at[(j + 4) % RING], isem)
            pltpu.make_async_copy(colk_hbm.at[base + g], cidx.at[j],
                                  isem).wait()
            pltpu.async_copy(ones_v, accum.at[cidx.at[j]], ssem, add=True)

    pltpu.make_async_copy(ones_v, accum.at[cidx.at[0]], ssem).wait()
    pltpu.make_async_copy(ones_v, accum.at[cidx.at[0]], ssem).wait()
    plsc.subcore_barrier()
    out = [out0_hbm, out1_hbm]
    for cc in range(NC):
        @pl.when(c == cc)
        def _():
            pltpu.sync_copy(accum.at[pl.ds(s * STRIPE, STRIPE)],
                            out[cc].at[pl.ds(s * STRIPE, STRIPE)])


def _pipelined_gather_scatter(table_hbm, rowk_hbm, colk_hbm, rbase, cbase,
                              ridx, cidx, vbufs, accum, isem, gsem, ssem,
                              ngc):
    for j in range(4):
        pltpu.async_copy(rowk_hbm.at[rbase + j], ridx.at[j], isem)
        pltpu.async_copy(colk_hbm.at[cbase + j], cidx.at[j], isem)
    for j in range(2):
        pltpu.make_async_copy(rowk_hbm.at[rbase], ridx.at[j], isem).wait()
        pltpu.make_async_copy(colk_hbm.at[cbase], cidx.at[j], isem).wait()
        pltpu.async_copy(table_hbm.at[ridx.at[j]], vbufs[j], gsem)

    @pl.loop(0, ngc // RING)
    def _(u):
        for j in range(RING):
            g = u * RING + j
            vb = vbufs[j % VR]

            @pl.when(g >= 2)
            def _():
                pltpu.make_async_copy(vb, accum.at[cidx.at[j]], ssem).wait()

            @pl.when(g + 4 < ngc)
            def _():
                jp = (j + 4) % RING
                pltpu.async_copy(rowk_hbm.at[rbase + g + 4], ridx.at[jp], isem)
                pltpu.async_copy(colk_hbm.at[cbase + g + 4], cidx.at[jp], isem)

            @pl.when(g + 2 < ngc)
            def _():
                jg = (j + 2) % RING
                pltpu.make_async_copy(rowk_hbm.at[rbase], ridx.at[jg],
                                      isem).wait()
                pltpu.make_async_copy(colk_hbm.at[cbase], cidx.at[jg],
                                      isem).wait()
                pltpu.async_copy(table_hbm.at[ridx.at[jg]],
                                 vbufs[(j + 2) % VR], gsem)
            pltpu.make_async_copy(table_hbm.at[ridx.at[j]], vb, gsem).wait()
            pltpu.async_copy(vb, accum.at[cidx.at[j]], ssem, add=True)

    pltpu.make_async_copy(vbufs[0], accum.at[cidx.at[0]], ssem).wait()
    pltpu.make_async_copy(vbufs[0], accum.at[cidx.at[0]], ssem).wait()


def _make_seg_kernel(width, ngc, two_core_scan):
    @functools.partial(
        pl.kernel,
        out_type=_out2(width),
        mesh=_mesh,
        compiler_params=_sc_params,
        scratch_types=[
            pltpu.VMEM((RING, CH), jnp.int32),
            pltpu.VMEM((RING, CH), jnp.int32),
            pltpu.VMEM((CH, width), jnp.float32),
            pltpu.VMEM((CH, width), jnp.float32),
            pltpu.VMEM((CH, width), jnp.float32),
            pltpu.VMEM((CH, width), jnp.float32),
            pltpu.VMEM_SHARED((N_PAD, width), jnp.float32),
            pltpu.SemaphoreType.DMA,
            pltpu.SemaphoreType.DMA,
            pltpu.SemaphoreType.DMA,
        ],
    )
    def seg(table_hbm, rowk_hbm, colk_hbm, zeros_hbm, out0_hbm, out1_hbm,
            ridx, cidx, v0, v1, v2, v3, accum, isem, gsem, ssem):
        c = lax.axis_index("c")
        s = lax.axis_index("s")
        if two_core_scan:
            rbase = c * K + s * ngc
            cbase = s * ngc
        else:
            rbase = (c * NS + s) * ngc
            cbase = rbase
        pltpu.sync_copy(zeros_hbm, accum.at[pl.ds(s * STRIPE, STRIPE)])
        plsc.subcore_barrier()
        _pipelined_gather_scatter(table_hbm, rowk_hbm, colk_hbm, rbase, cbase,
                                  ridx, cidx, (v0, v1, v2, v3),
                                  accum, isem, gsem, ssem, ngc)
        plsc.subcore_barrier()
        out = [out0_hbm, out1_hbm]
        for cc in range(NC):
            @pl.when(c == cc)
            def _():
                pltpu.sync_copy(accum.at[pl.ds(s * STRIPE, STRIPE)],
                                out[cc].at[pl.ds(s * STRIPE, STRIPE)])

    return seg


_sc_t = _make_seg_kernel(16, NGC2, False)
_sc_seg = _make_seg_kernel(H, NGC3, True)


NB = 3584
PBD = NB * 16 // 128


def _tc_dis_body(p0_ref, p1_ref, dpack_ref):
    deg = p0_ref[...] + p1_ref[...]
    dpack_ref[...] = jnp.where(deg > 0, lax.rsqrt(deg), 0.0)


_tc_dis = pl.pallas_call(
    _tc_dis_body,
    grid=(N_PAD // NB,),
    in_specs=[pl.BlockSpec((PBD, 128), lambda i: (i, 0)),
              pl.BlockSpec((PBD, 128), lambda i: (i, 0))],
    out_specs=pl.BlockSpec((PBD, 128), lambda i: (i, 0)),
    out_shape=jax.ShapeDtypeStruct((N_PAD * 16 // 128, 128), jnp.float32),
)


def _tc_prep_body(p0_ref, p1_ref, d_ref, x0_ref, s16_ref, y_ref):
    t = p0_ref[...] + p1_ref[...]
    d16 = d_ref[...]
    s16_ref[...] = d16 * t
    y_ref[...] = d16[:, 0:1] * x0_ref[...]


_tc_prep = pl.pallas_call(
    _tc_prep_body,
    grid=(N_PAD // NB,),
    in_specs=[
        pl.BlockSpec((NB, 16), lambda i: (i, 0)),
        pl.BlockSpec((NB, 16), lambda i: (i, 0)),
        pl.BlockSpec((NB, 16), lambda i: (i, 0)),
        pl.BlockSpec((NB, D), lambda i: (i, 0)),
    ],
    out_specs=[
        pl.BlockSpec((NB, 16), lambda i: (i, 0)),
        pl.BlockSpec((NB, D), lambda i: (i, 0)),
    ],
    out_shape=[
        jax.ShapeDtypeStruct((N_PAD, 16), jnp.float32),
        jax.ShapeDtypeStruct((N_PAD, D), jnp.float32),
    ],
)


def _tc_layer_body(g0_ref, g1_ref, d_ref, x_ref, s16_ref, w1_ref, w2_ref,
                   b_ref, xn_ref, y_ref):
    graw = jnp.concatenate([g0_ref[...], g1_ref[...]], axis=1)
    d1 = d_ref[...][:, 0:1]
    a = d1 * graw
    x = x_ref[...]
    acc = lax.dot_general(a, w1_ref[...], (((1,), (1,)), ((), ())),
                          preferred_element_type=jnp.float32)
    acc = acc + lax.dot_general(x * a, w2_ref[...], (((1,), (1,)), ((), ())),
                                preferred_element_type=jnp.float32)
    acc = acc + s16_ref[...][:, 0:1] * b_ref[...]
    xn = jnp.where(acc >= 0, acc, 0.01 * acc)
    xn_ref[...] = xn
    y_ref[...] = d1 * xn


_tc_layer = pl.pallas_call(
    _tc_layer_body,
    grid=(N_PAD // NB,),
    in_specs=[
        pl.BlockSpec((NB, H), lambda i: (i, 0)),
        pl.BlockSpec((NB, H), lambda i: (i, 0)),
        pl.BlockSpec((NB, 16), lambda i: (i, 0)),
        pl.BlockSpec((NB, D), lambda i: (i, 0)),
        pl.BlockSpec((NB, 16), lambda i: (i, 0)),
        pl.BlockSpec((D, D), lambda i: (0, 0)),
        pl.BlockSpec((D, D), lambda i: (0, 0)),
        pl.BlockSpec((1, D), lambda i: (0, 0)),
    ],
    out_specs=[
        pl.BlockSpec((NB, D), lambda i: (i, 0)),
        pl.BlockSpec((NB, D), lambda i: (i, 0)),
    ],
    out_shape=[
        jax.ShapeDtypeStruct((N_PAD, D), jnp.float32),
        jax.ShapeDtypeStruct((N_PAD, D), jnp.float32),
    ],
)


def _to_sc_table(y, rows, width):
    pack = lax.optimization_barrier(y.reshape(rows * width // 128, 128))
    return pack.reshape(rows, width)


def kernel(user_w, item_w, W1_0, b1_0, W2_0, b2_0, W1_1, b1_1, W2_1, b2_1,
           W1_2, b1_2, W2_2, b2_2, edge_index):
    row = edge_index[0].astype(jnp.int32)
    col = edge_index[1].astype(jnp.int32)
    pad = E_PAD - E
    colp = jnp.concatenate([col, jnp.full((pad,), N, jnp.int32)])
    rowp = jnp.concatenate([row, jnp.zeros((pad,), jnp.int32)])
    colk = colp.reshape(K, CH)
    rowk = rowp.reshape(K, CH)
    rowy = jnp.concatenate([2 * rowp, 2 * rowp + 1]).reshape(2 * K, CH)

    ones16 = jnp.ones((16, CH), jnp.float32).reshape(CH, 16)
    zeros16 = jnp.zeros((STRIPE * 16 // 128, 128), jnp.float32)
    zeros32 = jnp.zeros((STRIPE * H // 128, 128), jnp.float32)
    x0 = jnp.concatenate([user_w, item_w,
                          jnp.zeros((N_PAD - N, D), jnp.float32)], axis=0)

    dp0, dp1 = _sc_deg(colk, ones16, zeros16.reshape(STRIPE, 16))
    dpack = _tc_dis(dp0.reshape(N_PAD * 16 // 128, 128),
                    dp1.reshape(N_PAD * 16 // 128, 128))
    distab = dpack.reshape(N_PAD, 16)
    dis16 = dpack.reshape(N_PAD, 16)
    tp0, tp1 = _sc_t(distab, rowk, colk, zeros16.reshape(STRIPE, 16))
    s16, y = _tc_prep(tp0, tp1, dis16, x0)

    params = [(W1_0, b1_0, W2_0, b2_0), (W1_1, b1_1, W2_1, b2_1),
              (W1_2, b1_2, W2_2, b2_2)]
    embs = [x0]
    x = x0
    for (W1, b1, W2, b2) in params:
        ytab = _to_sc_table(y, 2 * N_PAD, H)
        gp0, gp1 = _sc_seg(ytab, rowy, colk, zeros32.reshape(STRIPE, H))
        bsum = (b1 + b2).reshape(1, D)
        x, y = _tc_layer(gp0, gp1, dis16, x, s16, W1, W2, bsum)
        embs.append(x)

    out = jnp.concatenate(embs, axis=1)
    return out[:N_USERS], out[N_USERS:N]


# speedup vs baseline: 2.0466x; 2.0466x over previous
import functools

import jax
import jax.numpy as jnp
from jax import lax
from jax.experimental import pallas as pl
from jax.experimental.pallas import tpu as pltpu
from jax.experimental.pallas import tpu_sc as plsc

N_USERS = 25000
N_ITEMS = 25000
N = N_USERS + N_ITEMS
E = 800000
D = 64
H = 32

NC = 2
NS = 16
CH = 128

N_PAD = 50176
STRIPE = N_PAD // NS
E_PAD = 819200
K = E_PAD // CH
NGC2 = K // (NC * NS)
NGC3 = K // NS
RING = 8
VR = 4

_mesh = plsc.VectorSubcoreMesh(core_axis_name="c", subcore_axis_name="s")
_sc_params = pltpu.CompilerParams(use_tc_tiling_on_sc=False)


def _out2(width):
    return [jax.ShapeDtypeStruct((N_PAD, width), jnp.float32),
            jax.ShapeDtypeStruct((N_PAD, width), jnp.float32)]


@functools.partial(
    pl.kernel,
    out_type=_out2(16),
    mesh=_mesh,
    compiler_params=_sc_params,
    scratch_types=[
        pltpu.VMEM((RING, CH), jnp.int32),
        pltpu.VMEM((CH, 16), jnp.float32),
        pltpu.VMEM_SHARED((N_PAD, 16), jnp.float32),
        pltpu.SemaphoreType.DMA,
        pltpu.SemaphoreType.DMA,
    ],
)
def _sc_deg(colk_hbm, ones_hbm, zeros_hbm, out0_hbm, out1_hbm,
            cidx, ones_v, accum, isem, ssem):
    c = lax.axis_index("c")
    s = lax.axis_index("s")
    base = (c * NS + s) * NGC2
    for j in range(4):
        pltpu.async_copy(colk_hbm.at[base + j], cidx.at[j], isem)
    pltpu.sync_copy(ones_hbm, ones_v)
    pltpu.sync_copy(zeros_hbm, accum.at[pl.ds(s * STRIPE, STRIPE)])
    plsc.subcore_barrier()

    @pl.loop(0, NGC2 // RING)
    def _(u):
        for j in range(RING):
            g = u * RING + j

            @pl.when(g >= 2)
            def _():
                pltpu.make_async_copy(ones_v, accum.at[cidx.at[j]], ssem).wait()

            @pl.when(g + 4 < NGC2)
            def _():
                pltpu.async_copy(colk_hbm.at[base + g + 4],
                                 cidx.at[(j + 4) % RING], isem)
            pltpu.make_async_copy(colk_hbm.at[base + g], cidx.at[j],
                                  isem).wait()
            pltpu.async_copy(ones_v, accum.at[cidx.at[j]], ssem, add=True)

    pltpu.make_async_copy(ones_v, accum.at[cidx.at[0]], ssem).wait()
    pltpu.make_async_copy(ones_v, accum.at[cidx.at[0]], ssem).wait()
    plsc.subcore_barrier()
    out = [out0_hbm, out1_hbm]
    for cc in range(NC):
        @pl.when(c == cc)
        def _():
            pltpu.sync_copy(accum.at[pl.ds(s * STRIPE, STRIPE)],
                            out[cc].at[pl.ds(s * STRIPE, STRIPE)])


def _pipelined_gather_scatter(table_hbm, rowk_hbm, colk_hbm, rbase, cbase,
                              ridx, cidx, vbufs, accum, isem, gsem, ssem,
                              ngc):
    for j in range(4):
        pltpu.async_copy(rowk_hbm.at[rbase + j], ridx.at[j], isem)
        pltpu.async_copy(colk_hbm.at[cbase + j], cidx.at[j], isem)
    for j in range(2):
        pltpu.make_async_copy(rowk_hbm.at[rbase], ridx.at[j], isem).wait()
        pltpu.make_async_copy(colk_hbm.at[cbase], cidx.at[j], isem).wait()
        pltpu.async_copy(table_hbm.at[ridx.at[j]], vbufs[j], gsem)

    @pl.loop(0, ngc // RING)
    def _(u):
        for j in range(RING):
            g = u * RING + j
            vb = vbufs[j % VR]

            @pl.when(g >= 2)
            def _():
                pltpu.make_async_copy(vb, accum.at[cidx.at[j]], ssem).wait()

            @pl.when(g + 4 < ngc)
            def _():
                jp = (j + 4) % RING
                pltpu.async_copy(rowk_hbm.at[rbase + g + 4], ridx.at[jp], isem)
                pltpu.async_copy(colk_hbm.at[cbase + g + 4], cidx.at[jp], isem)

            @pl.when(g + 2 < ngc)
            def _():
                jg = (j + 2) % RING
                pltpu.make_async_copy(rowk_hbm.at[rbase], ridx.at[jg],
                                      isem).wait()
                pltpu.make_async_copy(colk_hbm.at[cbase], cidx.at[jg],
                                      isem).wait()
                pltpu.async_copy(table_hbm.at[ridx.at[jg]],
                                 vbufs[(j + 2) % VR], gsem)
            pltpu.make_async_copy(table_hbm.at[ridx.at[j]], vb, gsem).wait()
            pltpu.async_copy(vb, accum.at[cidx.at[j]], ssem, add=True)

    pltpu.make_async_copy(vbufs[0], accum.at[cidx.at[0]], ssem).wait()
    pltpu.make_async_copy(vbufs[0], accum.at[cidx.at[0]], ssem).wait()


def _make_seg_kernel(width, ngc, two_core_scan):
    @functools.partial(
        pl.kernel,
        out_type=_out2(width),
        mesh=_mesh,
        compiler_params=_sc_params,
        scratch_types=[
            pltpu.VMEM((RING, CH), jnp.int32),
            pltpu.VMEM((RING, CH), jnp.int32),
            pltpu.VMEM((CH, width), jnp.float32),
            pltpu.VMEM((CH, width), jnp.float32),
            pltpu.VMEM((CH, width), jnp.float32),
            pltpu.VMEM((CH, width), jnp.float32),
            pltpu.VMEM_SHARED((N_PAD, width), jnp.float32),
            pltpu.SemaphoreType.DMA,
            pltpu.SemaphoreType.DMA,
            pltpu.SemaphoreType.DMA,
        ],
    )
    def seg(table_hbm, rowk_hbm, colk_hbm, zeros_hbm, out0_hbm, out1_hbm,
            ridx, cidx, v0, v1, v2, v3, accum, isem, gsem, ssem):
        c = lax.axis_index("c")
        s = lax.axis_index("s")
        if two_core_scan:
            rbase = c * K + s * ngc
            cbase = s * ngc
        else:
            rbase = (c * NS + s) * ngc
            cbase = rbase
        pltpu.sync_copy(zeros_hbm, accum.at[pl.ds(s * STRIPE, STRIPE)])
        plsc.subcore_barrier()
        _pipelined_gather_scatter(table_hbm, rowk_hbm, colk_hbm, rbase, cbase,
                                  ridx, cidx, (v0, v1, v2, v3),
                                  accum, isem, gsem, ssem, ngc)
        plsc.subcore_barrier()
        out = [out0_hbm, out1_hbm]
        for cc in range(NC):
            @pl.when(c == cc)
            def _():
                pltpu.sync_copy(accum.at[pl.ds(s * STRIPE, STRIPE)],
                                out[cc].at[pl.ds(s * STRIPE, STRIPE)])

    return seg


_sc_t = _make_seg_kernel(16, NGC2, False)
_sc_seg = _make_seg_kernel(H, NGC3, True)


NB = 3584
PBD = NB * 16 // 128


def _tc_dis_body(p0_ref, p1_ref, dpack_ref):
    deg = p0_ref[...] + p1_ref[...]
    dpack_ref[...] = jnp.where(deg > 0, lax.rsqrt(deg), 0.0)


_tc_dis = pl.pallas_call(
    _tc_dis_body,
    grid=(N_PAD // NB,),
    in_specs=[pl.BlockSpec((PBD, 128), lambda i: (i, 0)),
              pl.BlockSpec((PBD, 128), lambda i: (i, 0))],
    out_specs=pl.BlockSpec((PBD, 128), lambda i: (i, 0)),
    out_shape=jax.ShapeDtypeStruct((N_PAD * 16 // 128, 128), jnp.float32),
)


def _tc_prep_body(p0_ref, p1_ref, d_ref, x0_ref, s16_ref, y_ref):
    t = p0_ref[...] + p1_ref[...]
    d16 = d_ref[...]
    s16_ref[...] = d16 * t
    y_ref[...] = d16[:, 0:1] * x0_ref[...]


_tc_prep = pl.pallas_call(
    _tc_prep_body,
    grid=(N_PAD // NB,),
    in_specs=[
        pl.BlockSpec((NB, 16), lambda i: (i, 0)),
        pl.BlockSpec((NB, 16), lambda i: (i, 0)),
        pl.BlockSpec((NB, 16), lambda i: (i, 0)),
        pl.BlockSpec((NB, D), lambda i: (i, 0)),
    ],
    out_specs=[
        pl.BlockSpec((NB, 16), lambda i: (i, 0)),
        pl.BlockSpec((NB, D), lambda i: (i, 0)),
    ],
    out_shape=[
        jax.ShapeDtypeStruct((N_PAD, 16), jnp.float32),
        jax.ShapeDtypeStruct((N_PAD, D), jnp.float32),
    ],
)


def _tc_layer_body(g0_ref, g1_ref, d_ref, x_ref, s16_ref, w1_ref, w2_ref,
                   b_ref, xn_ref, y_ref):
    graw = jnp.concatenate([g0_ref[...], g1_ref[...]], axis=1)
    d1 = d_ref[...][:, 0:1]
    a = d1 * graw
    x = x_ref[...]
    acc = lax.dot_general(a, w1_ref[...], (((1,), (1,)), ((), ())),
                          preferred_element_type=jnp.float32)
    acc = acc + lax.dot_general(x * a, w2_ref[...], (((1,), (1,)), ((), ())),
                                preferred_element_type=jnp.float32)
    acc = acc + s16_ref[...][:, 0:1] * b_ref[...]
    xn = jnp.where(acc >= 0, acc, 0.01 * acc)
    xn_ref[...] = xn
    y_ref[...] = d1 * xn


_tc_layer = pl.pallas_call(
    _tc_layer_body,
    grid=(N_PAD // NB,),
    in_specs=[
        pl.BlockSpec((NB, H), lambda i: (i, 0)),
        pl.BlockSpec((NB, H), lambda i: (i, 0)),
        pl.BlockSpec((NB, 16), lambda i: (i, 0)),
        pl.BlockSpec((NB, D), lambda i: (i, 0)),
        pl.BlockSpec((NB, 16), lambda i: (i, 0)),
        pl.BlockSpec((D, D), lambda i: (0, 0)),
        pl.BlockSpec((D, D), lambda i: (0, 0)),
        pl.BlockSpec((1, D), lambda i: (0, 0)),
    ],
    out_specs=[
        pl.BlockSpec((NB, D), lambda i: (i, 0)),
        pl.BlockSpec((NB, D), lambda i: (i, 0)),
    ],
    out_shape=[
        jax.ShapeDtypeStruct((N_PAD, D), jnp.float32),
        jax.ShapeDtypeStruct((N_PAD, D), jnp.float32),
    ],
)


def _to_sc_table(y, rows, width):
    pack = lax.optimization_barrier(y.reshape(rows * width // 128, 128))
    return pack.reshape(rows, width)


def kernel(user_w, item_w, W1_0, b1_0, W2_0, b2_0, W1_1, b1_1, W2_1, b2_1,
           W1_2, b1_2, W2_2, b2_2, edge_index):
    row = edge_index[0].astype(jnp.int32)
    col = edge_index[1].astype(jnp.int32)
    pad = E_PAD - E
    pad_iota = lax.iota(jnp.int32, pad)
    colp = jnp.concatenate([col, N + pad_iota % (N_PAD - N)])
    rowp = jnp.concatenate([row, pad_iota % N])
    colk = colp.reshape(K, CH)
    rowk = rowp.reshape(K, CH)
    rowy = jnp.concatenate([2 * rowp, 2 * rowp + 1]).reshape(2 * K, CH)

    ones16 = jnp.ones((16, CH), jnp.float32).reshape(CH, 16)
    zeros16 = jnp.zeros((STRIPE * 16 // 128, 128), jnp.float32)
    zeros32 = jnp.zeros((STRIPE * H // 128, 128), jnp.float32)
    x0 = jnp.concatenate([user_w, item_w,
                          jnp.zeros((N_PAD - N, D), jnp.float32)], axis=0)

    dp0, dp1 = _sc_deg(colk, ones16, zeros16.reshape(STRIPE, 16))
    dpack = _tc_dis(dp0.reshape(N_PAD * 16 // 128, 128),
                    dp1.reshape(N_PAD * 16 // 128, 128))
    distab = dpack.reshape(N_PAD, 16)
    dis16 = dpack.reshape(N_PAD, 16)
    tp0, tp1 = _sc_t(distab, rowk, colk, zeros16.reshape(STRIPE, 16))
    s16, y = _tc_prep(tp0, tp1, dis16, x0)

    params = [(W1_0, b1_0, W2_0, b2_0), (W1_1, b1_1, W2_1, b2_1),
              (W1_2, b1_2, W2_2, b2_2)]
    embs = [x0]
    x = x0
    for (W1, b1, W2, b2) in params:
        ytab = _to_sc_table(y, 2 * N_PAD, H)
        gp0, gp1 = _sc_seg(ytab, rowy, colk, zeros32.reshape(STRIPE, H))
        bsum = (b1 + b2).reshape(1, D)
        x, y = _tc_layer(gp0, gp1, dis16, x, s16, W1, W2, bsum)
        embs.append(x)

    out = jnp.concatenate(embs, axis=1)
    return out[:N_USERS], out[N_USERS:N]


# speedup vs baseline: 2.0639x; 1.0085x over previous
import functools

import jax
import jax.numpy as jnp
from jax import lax
from jax.experimental import pallas as pl
from jax.experimental.pallas import tpu as pltpu
from jax.experimental.pallas import tpu_sc as plsc

N_USERS = 25000
N_ITEMS = 25000
N = N_USERS + N_ITEMS
E = 800000
D = 64
H = 32

NC = 2
NS = 16
CH = 128

N_PAD = 50176
STRIPE = N_PAD // NS
E_PAD = 819200
K = E_PAD // CH
NGC2 = K // (NC * NS)
NGC3 = K // NS
RING = 8
VR = 4

_mesh = plsc.VectorSubcoreMesh(core_axis_name="c", subcore_axis_name="s")
_sc_params = pltpu.CompilerParams(use_tc_tiling_on_sc=False)


def _out2(width):
    return [jax.ShapeDtypeStruct((N_PAD, width), jnp.float32),
            jax.ShapeDtypeStruct((N_PAD, width), jnp.float32)]


@functools.partial(
    pl.kernel,
    out_type=_out2(16),
    mesh=_mesh,
    compiler_params=_sc_params,
    scratch_types=[
        pltpu.VMEM((RING, CH), jnp.int32),
        pltpu.VMEM((CH, 16), jnp.float32),
        pltpu.VMEM_SHARED((N_PAD, 16), jnp.float32),
        pltpu.SemaphoreType.DMA,
        pltpu.SemaphoreType.DMA,
    ],
)
def _sc_deg(colk_hbm, ones_hbm, zeros_hbm, out0_hbm, out1_hbm,
            cidx, ones_v, accum, isem, ssem):
    c = lax.axis_index("c")
    s = lax.axis_index("s")
    base = (c * NS + s) * NGC2
    for j in range(4):
        pltpu.async_copy(colk_hbm.at[base + j], cidx.at[j], isem)
    pltpu.sync_copy(ones_hbm, ones_v)
    pltpu.sync_copy(zeros_hbm, accum.at[pl.ds(s * STRIPE, STRIPE)])
    plsc.subcore_barrier()

    @pl.loop(0, NGC2 // RING)
    def _(u):
        for j in range(RING):
            g = u * RING + j

            @pl.when(g >= 2)
            def _():
                pltpu.make_async_copy(ones_v, accum.at[cidx.at[j]], ssem).wait()

            @pl.when(g + 4 < NGC2)
            def _():
                pltpu.async_copy(colk_hbm.at[base + g + 4],
                                 cidx.at[(j + 4) % RING], isem)
            pltpu.make_async_copy(colk_hbm.at[base + g], cidx.at[j],
                                  isem).wait()
            pltpu.async_copy(ones_v, accum.at[cidx.at[j]], ssem, add=True)

    pltpu.make_async_copy(ones_v, accum.at[cidx.at[0]], ssem).wait()
    pltpu.make_async_copy(ones_v, accum.at[cidx.at[0]], ssem).wait()
    plsc.subcore_barrier()
    out = [out0_hbm, out1_hbm]
    for cc in range(NC):
        @pl.when(c == cc)
        def _():
            pltpu.sync_copy(accum.at[pl.ds(s * STRIPE, STRIPE)],
                            out[cc].at[pl.ds(s * STRIPE, STRIPE)])


def _pipelined_gather_scatter(table_hbm, rowk_hbm, colk_hbm, rbase, cbase,
                              ridx, cidx, vbufs, accum, isem, gsem, ssem,
                              ngc):
    for j in range(4):
        pltpu.async_copy(rowk_hbm.at[rbase + j], ridx.at[j], isem)
        pltpu.async_copy(colk_hbm.at[cbase + j], cidx.at[j], isem)
    for j in range(2):
        pltpu.make_async_copy(rowk_hbm.at[rbase], ridx.at[j], isem).wait()
        pltpu.make_async_copy(colk_hbm.at[cbase], cidx.at[j], isem).wait()
        pltpu.async_copy(table_hbm.at[ridx.at[j]], vbufs[j], gsem)

    @pl.loop(0, ngc // RING)
    def _(u):
        for j in range(RING):
            g = u * RING + j
            vb = vbufs[j % VR]

            @pl.when(g >= 2)
            def _():
                pltpu.make_async_copy(vb, accum.at[cidx.at[j]], ssem).wait()

            @pl.when(g + 4 < ngc)
            def _():
                jp = (j + 4) % RING
                pltpu.async_copy(rowk_hbm.at[rbase + g + 4], ridx.at[jp], isem)
                pltpu.async_copy(colk_hbm.at[cbase + g + 4], cidx.at[jp], isem)

            @pl.when(g + 2 < ngc)
            def _():
                jg = (j + 2) % RING
                pltpu.make_async_copy(rowk_hbm.at[rbase], ridx.at[jg],
                                      isem).wait()
                pltpu.make_async_copy(colk_hbm.at[cbase], cidx.at[jg],
                                      isem).wait()
                pltpu.async_copy(table_hbm.at[ridx.at[jg]],
                                 vbufs[(j + 2) % VR], gsem)
            pltpu.make_async_copy(table_hbm.at[ridx.at[j]], vb, gsem).wait()
            pltpu.async_copy(vb, accum.at[cidx.at[j]], ssem, add=True)

    pltpu.make_async_copy(vbufs[0], accum.at[cidx.at[0]], ssem).wait()
    pltpu.make_async_copy(vbufs[0], accum.at[cidx.at[0]], ssem).wait()


def _make_seg_kernel(width, ngc, two_core_scan):
    @functools.partial(
        pl.kernel,
        out_type=_out2(width),
        mesh=_mesh,
        compiler_params=_sc_params,
        scratch_types=[
            pltpu.VMEM((RING, CH), jnp.int32),
            pltpu.VMEM((RING, CH), jnp.int32),
            pltpu.VMEM((CH, width), jnp.float32),
            pltpu.VMEM((CH, width), jnp.float32),
            pltpu.VMEM((CH, width), jnp.float32),
            pltpu.VMEM((CH, width), jnp.float32),
            pltpu.VMEM_SHARED((N_PAD, width), jnp.float32),
            pltpu.SemaphoreType.DMA,
            pltpu.SemaphoreType.DMA,
            pltpu.SemaphoreType.DMA,
        ],
    )
    def seg(table_hbm, rowk_hbm, colk_hbm, zeros_hbm, out0_hbm, out1_hbm,
            ridx, cidx, v0, v1, v2, v3, accum, isem, gsem, ssem):
        c = lax.axis_index("c")
        s = lax.axis_index("s")
        if two_core_scan:
            rbase = c * K + s * ngc
            cbase = s * ngc
        else:
            rbase = (c * NS + s) * ngc
            cbase = rbase
        pltpu.sync_copy(zeros_hbm, accum.at[pl.ds(s * STRIPE, STRIPE)])
        plsc.subcore_barrier()
        _pipelined_gather_scatter(table_hbm, rowk_hbm, colk_hbm, rbase, cbase,
                                  ridx, cidx, (v0, v1, v2, v3),
                                  accum, isem, gsem, ssem, ngc)
        plsc.subcore_barrier()
        out = [out0_hbm, out1_hbm]
        for cc in range(NC):
            @pl.when(c == cc)
            def _():
                pltpu.sync_copy(accum.at[pl.ds(s * STRIPE, STRIPE)],
                                out[cc].at[pl.ds(s * STRIPE, STRIPE)])

    return seg


_sc_t = _make_seg_kernel(16, NGC2, False)
_sc_seg = _make_seg_kernel(H, NGC3, True)


NB = 3584
PBD = NB * 16 // 128


def _tc_dis_body(p0_ref, p1_ref, dpack_ref):
    deg = p0_ref[...] + p1_ref[...]
    dpack_ref[...] = jnp.where(deg > 0, lax.rsqrt(deg), 0.0)


_tc_dis = pl.pallas_call(
    _tc_dis_body,
    grid=(N_PAD // NB,),
    in_specs=[pl.BlockSpec((PBD, 128), lambda i: (i, 0)),
              pl.BlockSpec((PBD, 128), lambda i: (i, 0))],
    out_specs=pl.BlockSpec((PBD, 128), lambda i: (i, 0)),
    out_shape=jax.ShapeDtypeStruct((N_PAD * 16 // 128, 128), jnp.float32),
)


def _tc_prep_body(p0_ref, p1_ref, d_ref, x0_ref, s16_ref, y_ref):
    t = p0_ref[...] + p1_ref[...]
    d16 = d_ref[...]
    s16_ref[...] = d16 * t
    y_ref[...] = d16[:, 0:1] * x0_ref[...]


_tc_prep = pl.pallas_call(
    _tc_prep_body,
    grid=(N_PAD // NB,),
    in_specs=[
        pl.BlockSpec((NB, 16), lambda i: (i, 0)),
        pl.BlockSpec((NB, 16), lambda i: (i, 0)),
        pl.BlockSpec((NB, 16), lambda i: (i, 0)),
        pl.BlockSpec((NB, D), lambda i: (i, 0)),
    ],
    out_specs=[
        pl.BlockSpec((NB, 16), lambda i: (i, 0)),
        pl.BlockSpec((NB, D), lambda i: (i, 0)),
    ],
    out_shape=[
        jax.ShapeDtypeStruct((N_PAD, 16), jnp.float32),
        jax.ShapeDtypeStruct((N_PAD, D), jnp.float32),
    ],
)


def _tc_layer_body(g0_ref, g1_ref, d_ref, x_ref, s16_ref, w1_ref, w2_ref,
                   b_ref, xn_ref, y_ref):
    graw = jnp.concatenate([g0_ref[...], g1_ref[...]], axis=1)
    d1 = d_ref[...][:, 0:1]
    a = d1 * graw
    x = x_ref[...]
    acc = lax.dot_general(a, w1_ref[...], (((1,), (1,)), ((), ())),
                          preferred_element_type=jnp.float32)
    acc = acc + lax.dot_general(x * a, w2_ref[...], (((1,), (1,)), ((), ())),
                                preferred_element_type=jnp.float32)
    acc = acc + s16_ref[...][:, 0:1] * b_ref[...]
    xn = jnp.where(acc >= 0, acc, 0.01 * acc)
    xn_ref[...] = xn
    y_ref[...] = d1 * xn


_tc_layer = pl.pallas_call(
    _tc_layer_body,
    grid=(N_PAD // NB,),
    in_specs=[
        pl.BlockSpec((NB, H), lambda i: (i, 0)),
        pl.BlockSpec((NB, H), lambda i: (i, 0)),
        pl.BlockSpec((NB, 16), lambda i: (i, 0)),
        pl.BlockSpec((NB, D), lambda i: (i, 0)),
        pl.BlockSpec((NB, 16), lambda i: (i, 0)),
        pl.BlockSpec((D, D), lambda i: (0, 0)),
        pl.BlockSpec((D, D), lambda i: (0, 0)),
        pl.BlockSpec((1, D), lambda i: (0, 0)),
    ],
    out_specs=[
        pl.BlockSpec((NB, D), lambda i: (i, 0)),
        pl.BlockSpec((NB, D), lambda i: (i, 0)),
    ],
    out_shape=[
        jax.ShapeDtypeStruct((N_PAD, D), jnp.float32),
        jax.ShapeDtypeStruct((N_PAD, D), jnp.float32),
    ],
)


NBA = 5000


def _asm_body(w_ref, x1_ref, x2_ref, x3_ref, o_ref):
    o_ref[...] = jnp.concatenate(
        [w_ref[...], x1_ref[...], x2_ref[...], x3_ref[...]], axis=1)


def _make_asm(row_off):
    blocks = N_USERS // NBA
    return pl.pallas_call(
        _asm_body,
        grid=(blocks,),
        in_specs=[
            pl.BlockSpec((NBA, D), lambda i: (i, 0)),
            pl.BlockSpec((NBA, D), lambda i, o=row_off // NBA: (i + o, 0)),
            pl.BlockSpec((NBA, D), lambda i, o=row_off // NBA: (i + o, 0)),
            pl.BlockSpec((NBA, D), lambda i, o=row_off // NBA: (i + o, 0)),
        ],
        out_specs=pl.BlockSpec((NBA, 4 * D), lambda i: (i, 0)),
        out_shape=jax.ShapeDtypeStruct((N_USERS, 4 * D), jnp.float32),
    )


_asm_user = _make_asm(0)
_asm_item = _make_asm(N_USERS)


def _to_sc_table(y, rows, width):
    pack = lax.optimization_barrier(y.reshape(rows * width // 128, 128))
    return pack.reshape(rows, width)


def kernel(user_w, item_w, W1_0, b1_0, W2_0, b2_0, W1_1, b1_1, W2_1, b2_1,
           W1_2, b1_2, W2_2, b2_2, edge_index):
    row = edge_index[0].astype(jnp.int32)
    col = edge_index[1].astype(jnp.int32)
    pad = E_PAD - E
    pad_iota = lax.iota(jnp.int32, pad)
    colp = jnp.concatenate([col, N + pad_iota % (N_PAD - N)])
    rowp = jnp.concatenate([row, pad_iota % N])
    colk = colp.reshape(K, CH)
    rowk = rowp.reshape(K, CH)
    rowy = jnp.concatenate([2 * rowp, 2 * rowp + 1]).reshape(2 * K, CH)

    ones16 = jnp.ones((16, CH), jnp.float32).reshape(CH, 16)
    zeros16 = jnp.zeros((STRIPE * 16 // 128, 128), jnp.float32)
    zeros32 = jnp.zeros((STRIPE * H // 128, 128), jnp.float32)
    x0 = jnp.concatenate([user_w, item_w,
                          jnp.zeros((N_PAD - N, D), jnp.float32)], axis=0)

    dp0, dp1 = _sc_deg(colk, ones16, zeros16.reshape(STRIPE, 16))
    dpack = _tc_dis(dp0.reshape(N_PAD * 16 // 128, 128),
                    dp1.reshape(N_PAD * 16 // 128, 128))
    distab = dpack.reshape(N_PAD, 16)
    dis16 = dpack.reshape(N_PAD, 16)
    tp0, tp1 = _sc_t(distab, rowk, colk, zeros16.reshape(STRIPE, 16))
    s16, y = _tc_prep(tp0, tp1, dis16, x0)

    params = [(W1_0, b1_0, W2_0, b2_0), (W1_1, b1_1, W2_1, b2_1),
              (W1_2, b1_2, W2_2, b2_2)]
    embs = [x0]
    x = x0
    for (W1, b1, W2, b2) in params:
        ytab = _to_sc_table(y, 2 * N_PAD, H)
        gp0, gp1 = _sc_seg(ytab, rowy, colk, zeros32.reshape(STRIPE, H))
        bsum = (b1 + b2).reshape(1, D)
        x, y = _tc_layer(gp0, gp1, dis16, x, s16, W1, W2, bsum)
        embs.append(x)

    x1, x2, x3 = embs[1], embs[2], embs[3]
    user_emb = _asm_user(user_w, x1, x2, x3)
    item_emb = _asm_item(item_w, x1, x2, x3)
    return user_emb, item_emb


# speedup vs baseline: 2.2013x; 1.0666x over previous
import functools

import jax
import jax.numpy as jnp
from jax import lax
from jax.experimental import pallas as pl
from jax.experimental.pallas import tpu as pltpu
from jax.experimental.pallas import tpu_sc as plsc

N_USERS = 25000
N_ITEMS = 25000
N = N_USERS + N_ITEMS
E = 800000
D = 64
H = 32

NC = 2
NS = 16
CH = 128

N_PAD = 50176
STRIPE = N_PAD // NS
E_PAD = 819200
K = E_PAD // CH
NGC2 = K // (NC * NS)
NGC3 = K // NS
RING = 8
VR = 4

_mesh = plsc.VectorSubcoreMesh(core_axis_name="c", subcore_axis_name="s")
_sc_params = pltpu.CompilerParams(use_tc_tiling_on_sc=False)


def _out2(width):
    return [jax.ShapeDtypeStruct((N_PAD, width), jnp.float32),
            jax.ShapeDtypeStruct((N_PAD, width), jnp.float32)]


@functools.partial(
    pl.kernel,
    out_type=_out2(16),
    mesh=_mesh,
    compiler_params=_sc_params,
    scratch_types=[
        pltpu.VMEM((RING, CH), jnp.int32),
        pltpu.VMEM((CH, 16), jnp.float32),
        pltpu.VMEM_SHARED((N_PAD, 16), jnp.float32),
        pltpu.SemaphoreType.DMA,
        pltpu.SemaphoreType.DMA,
    ],
)
def _sc_deg(colk_hbm, ones_hbm, zeros_hbm, out0_hbm, out1_hbm,
            cidx, ones_v, accum, isem, ssem):
    c = lax.axis_index("c")
    s = lax.axis_index("s")
    base = (c * NS + s) * NGC2
    for j in range(4):
        pltpu.async_copy(colk_hbm.at[base + j], cidx.at[j], isem)
    pltpu.sync_copy(ones_hbm, ones_v)
    pltpu.sync_copy(zeros_hbm, accum.at[pl.ds(s * STRIPE, STRIPE)])
    plsc.subcore_barrier()

    @pl.loop(0, NGC2 // RING)
    def _(u):
        for j in range(RING):
            g = u * RING + j

            @pl.when(g >= 2)
            def _():
                pltpu.make_async_copy(ones_v, accum.at[cidx.at[j]], ssem).wait()

            @pl.when(g + 4 < NGC2)
            def _():
                pltpu.async_copy(colk_hbm.at[base + g + 4],
                                 cidx.at[(j + 4) % RING], isem)
            pltpu.make_async_copy(colk_hbm.at[base + g], cidx.at[j],
                                  isem).wait()
            pltpu.async_copy(ones_v, accum.at[cidx.at[j]], ssem, add=True)

    pltpu.make_async_copy(ones_v, accum.at[cidx.at[0]], ssem).wait()
    pltpu.make_async_copy(ones_v, accum.at[cidx.at[0]], ssem).wait()
    plsc.subcore_barrier()
    out = [out0_hbm, out1_hbm]
    for cc in range(NC):
        @pl.when(c == cc)
        def _():
            pltpu.sync_copy(accum.at[pl.ds(s * STRIPE, STRIPE)],
                            out[cc].at[pl.ds(s * STRIPE, STRIPE)])


def _pipelined_gather_scatter(table_hbm, rowk_hbm, colk_hbm, rbase, cbase,
                              ridx, cidx, vbufs, accum, isem, gsem, ssem,
                              ngc):
    for j in range(4):
        pltpu.async_copy(rowk_hbm.at[rbase + j], ridx.at[j], isem)
        pltpu.async_copy(colk_hbm.at[cbase + j], cidx.at[j], isem)
    for j in range(2):
        pltpu.make_async_copy(rowk_hbm.at[rbase], ridx.at[j], isem).wait()
        pltpu.make_async_copy(colk_hbm.at[cbase], cidx.at[j], isem).wait()
        pltpu.async_copy(table_hbm.at[ridx.at[j]], vbufs[j], gsem)

    @pl.loop(0, ngc // RING)
    def _(u):
        for j in range(RING):
            g = u * RING + j
            vb = vbufs[j % VR]

            @pl.when(g >= 2)
            def _():
                pltpu.make_async_copy(vb, accum.at[cidx.at[j]], ssem).wait()

            @pl.when(g + 4 < ngc)
            def _():
                jp = (j + 4) % RING
                pltpu.async_copy(rowk_hbm.at[rbase + g + 4], ridx.at[jp], isem)
                pltpu.async_copy(colk_hbm.at[cbase + g + 4], cidx.at[jp], isem)

            @pl.when(g + 2 < ngc)
            def _():
                jg = (j + 2) % RING
                pltpu.make_async_copy(rowk_hbm.at[rbase], ridx.at[jg],
                                      isem).wait()
                pltpu.make_async_copy(colk_hbm.at[cbase], cidx.at[jg],
                                      isem).wait()
                pltpu.async_copy(table_hbm.at[ridx.at[jg]],
                                 vbufs[(j + 2) % VR], gsem)
            pltpu.make_async_copy(table_hbm.at[ridx.at[j]], vb, gsem).wait()
            pltpu.async_copy(vb, accum.at[cidx.at[j]], ssem, add=True)

    pltpu.make_async_copy(vbufs[0], accum.at[cidx.at[0]], ssem).wait()
    pltpu.make_async_copy(vbufs[0], accum.at[cidx.at[0]], ssem).wait()


def _make_seg_kernel(width, ngc, two_core_scan, wide_out=False):
    out_w = 128 if wide_out else width
    @functools.partial(
        pl.kernel,
        out_type=[jax.ShapeDtypeStruct((N_PAD, out_w), jnp.float32),
                  jax.ShapeDtypeStruct((N_PAD, out_w), jnp.float32)],
        mesh=_mesh,
        compiler_params=_sc_params,
        scratch_types=[
            pltpu.VMEM((RING, CH), jnp.int32),
            pltpu.VMEM((RING, CH), jnp.int32),
            pltpu.VMEM((CH, width), jnp.float32),
            pltpu.VMEM((CH, width), jnp.float32),
            pltpu.VMEM((CH, width), jnp.float32),
            pltpu.VMEM((CH, width), jnp.float32),
            pltpu.VMEM_SHARED((N_PAD, width), jnp.float32),
            pltpu.SemaphoreType.DMA,
            pltpu.SemaphoreType.DMA,
            pltpu.SemaphoreType.DMA,
        ],
    )
    def seg(table_hbm, rowk_hbm, colk_hbm, zeros_hbm, out0_hbm, out1_hbm,
            ridx, cidx, v0, v1, v2, v3, accum, isem, gsem, ssem):
        c = lax.axis_index("c")
        s = lax.axis_index("s")
        if two_core_scan:
            rbase = c * K + s * ngc
            cbase = s * ngc
        else:
            rbase = (c * NS + s) * ngc
            cbase = rbase
        pltpu.sync_copy(zeros_hbm, accum.at[pl.ds(s * STRIPE, STRIPE)])
        plsc.subcore_barrier()
        _pipelined_gather_scatter(table_hbm, rowk_hbm, colk_hbm, rbase, cbase,
                                  ridx, cidx, (v0, v1, v2, v3),
                                  accum, isem, gsem, ssem, ngc)
        plsc.subcore_barrier()
        out = [out0_hbm, out1_hbm]
        for cc in range(NC):
            @pl.when(c == cc)
            def _():
                if wide_out:
                    pltpu.sync_copy(
                        accum.at[pl.ds(s * STRIPE, STRIPE)],
                        out[cc].at[pl.ds(s * STRIPE, STRIPE), pl.ds(0, width)])
                else:
                    pltpu.sync_copy(accum.at[pl.ds(s * STRIPE, STRIPE)],
                                    out[cc].at[pl.ds(s * STRIPE, STRIPE)])

    return seg


_sc_t = _make_seg_kernel(16, NGC2, False)
_sc_seg = _make_seg_kernel(H, NGC3, True, wide_out=True)


NB = 3584
PBD = NB * 16 // 128


def _tc_dis_body(p0_ref, p1_ref, dpack_ref):
    deg = p0_ref[...] + p1_ref[...]
    dpack_ref[...] = jnp.where(deg > 0, lax.rsqrt(deg), 0.0)


_tc_dis = pl.pallas_call(
    _tc_dis_body,
    grid=(N_PAD // NB,),
    in_specs=[pl.BlockSpec((PBD, 128), lambda i: (i, 0)),
              pl.BlockSpec((PBD, 128), lambda i: (i, 0))],
    out_specs=pl.BlockSpec((PBD, 128), lambda i: (i, 0)),
    out_shape=jax.ShapeDtypeStruct((N_PAD * 16 // 128, 128), jnp.float32),
)


def _tc_prep_body(p0_ref, p1_ref, d_ref, x0_ref, s16_ref, y_ref):
    t = p0_ref[...] + p1_ref[...]
    d16 = d_ref[...]
    s16_ref[...] = d16 * t
    y_ref[...] = d16[:, 0:1] * x0_ref[...]


_tc_prep = pl.pallas_call(
    _tc_prep_body,
    grid=(N_PAD // NB,),
    in_specs=[
        pl.BlockSpec((NB, 16), lambda i: (i, 0)),
        pl.BlockSpec((NB, 16), lambda i: (i, 0)),
        pl.BlockSpec((NB, 16), lambda i: (i, 0)),
        pl.BlockSpec((NB, D), lambda i: (i, 0)),
    ],
    out_specs=[
        pl.BlockSpec((NB, 16), lambda i: (i, 0)),
        pl.BlockSpec((NB, D), lambda i: (i, 0)),
    ],
    out_shape=[
        jax.ShapeDtypeStruct((N_PAD, 16), jnp.float32),
        jax.ShapeDtypeStruct((N_PAD, D), jnp.float32),
    ],
)


def _tc_layer_body(g0_ref, g1_ref, d_ref, x_ref, s16_ref, w1_ref, w2_ref,
                   b_ref, xn_ref, y_ref):
    graw = jnp.concatenate([g0_ref[...][:, :H], g1_ref[...][:, :H]], axis=1)
    d1 = d_ref[...][:, 0:1]
    a = d1 * graw
    x = x_ref[...]
    acc = lax.dot_general(a, w1_ref[...], (((1,), (1,)), ((), ())),
                          preferred_element_type=jnp.float32)
    acc = acc + lax.dot_general(x * a, w2_ref[...], (((1,), (1,)), ((), ())),
                                preferred_element_type=jnp.float32)
    acc = acc + s16_ref[...][:, 0:1] * b_ref[...]
    xn = jnp.where(acc >= 0, acc, 0.01 * acc)
    xn_ref[...] = xn
    y_ref[...] = d1 * xn


_tc_layer = pl.pallas_call(
    _tc_layer_body,
    grid=(N_PAD // NB,),
    in_specs=[
        pl.BlockSpec((NB, 128), lambda i: (i, 0)),
        pl.BlockSpec((NB, 128), lambda i: (i, 0)),
        pl.BlockSpec((NB, 16), lambda i: (i, 0)),
        pl.BlockSpec((NB, D), lambda i: (i, 0)),
        pl.BlockSpec((NB, 16), lambda i: (i, 0)),
        pl.BlockSpec((D, D), lambda i: (0, 0)),
        pl.BlockSpec((D, D), lambda i: (0, 0)),
        pl.BlockSpec((1, D), lambda i: (0, 0)),
    ],
    out_specs=[
        pl.BlockSpec((NB, D), lambda i: (i, 0)),
        pl.BlockSpec((NB, D), lambda i: (i, 0)),
    ],
    out_shape=[
        jax.ShapeDtypeStruct((N_PAD, D), jnp.float32),
        jax.ShapeDtypeStruct((N_PAD, D), jnp.float32),
    ],
)


NBA = 5000


def _asm_body(w_ref, x1_ref, x2_ref, x3_ref, o_ref):
    o_ref[...] = jnp.concatenate(
        [w_ref[...], x1_ref[...], x2_ref[...], x3_ref[...]], axis=1)


def _make_asm(row_off):
    blocks = N_USERS // NBA
    return pl.pallas_call(
        _asm_body,
        grid=(blocks,),
        in_specs=[
            pl.BlockSpec((NBA, D), lambda i: (i, 0)),
            pl.BlockSpec((NBA, D), lambda i, o=row_off // NBA: (i + o, 0)),
            pl.BlockSpec((NBA, D), lambda i, o=row_off // NBA: (i + o, 0)),
            pl.BlockSpec((NBA, D), lambda i, o=row_off // NBA: (i + o, 0)),
        ],
        out_specs=pl.BlockSpec((NBA, 4 * D), lambda i: (i, 0)),
        out_shape=jax.ShapeDtypeStruct((N_USERS, 4 * D), jnp.float32),
    )


_asm_user = _make_asm(0)
_asm_item = _make_asm(N_USERS)


def _to_sc_table(y, rows, width):
    pack = lax.optimization_barrier(y.reshape(rows * width // 128, 128))
    return pack.reshape(rows, width)


def kernel(user_w, item_w, W1_0, b1_0, W2_0, b2_0, W1_1, b1_1, W2_1, b2_1,
           W1_2, b1_2, W2_2, b2_2, edge_index):
    row = edge_index[0].astype(jnp.int32)
    col = edge_index[1].astype(jnp.int32)
    pad = E_PAD - E
    pad_iota = lax.iota(jnp.int32, pad)
    colp = jnp.concatenate([col, N + pad_iota % (N_PAD - N)])
    rowp = jnp.concatenate([row, pad_iota % N])
    colk = colp.reshape(K, CH)
    rowk = rowp.reshape(K, CH)
    rowy = jnp.concatenate([2 * rowp, 2 * rowp + 1]).reshape(2 * K, CH)

    ones16 = jnp.ones((16, CH), jnp.float32).reshape(CH, 16)
    zeros16 = jnp.zeros((STRIPE * 16 // 128, 128), jnp.float32)
    zeros32 = jnp.zeros((STRIPE * H // 128, 128), jnp.float32)
    x0 = jnp.concatenate([user_w, item_w,
                          jnp.zeros((N_PAD - N, D), jnp.float32)], axis=0)

    dp0, dp1 = _sc_deg(colk, ones16, zeros16.reshape(STRIPE, 16))
    dpack = _tc_dis(dp0.reshape(N_PAD * 16 // 128, 128),
                    dp1.reshape(N_PAD * 16 // 128, 128))
    distab = dpack.reshape(N_PAD, 16)
    dis16 = dpack.reshape(N_PAD, 16)
    tp0, tp1 = _sc_t(distab, rowk, colk, zeros16.reshape(STRIPE, 16))
    s16, y = _tc_prep(tp0, tp1, dis16, x0)

    params = [(W1_0, b1_0, W2_0, b2_0), (W1_1, b1_1, W2_1, b2_1),
              (W1_2, b1_2, W2_2, b2_2)]
    embs = [x0]
    x = x0
    for (W1, b1, W2, b2) in params:
        ytab = _to_sc_table(y, 2 * N_PAD, H)
        gp0, gp1 = _sc_seg(ytab, rowy, colk, zeros32.reshape(STRIPE, H))
        bsum = (b1 + b2).reshape(1, D)
        x, y = _tc_layer(gp0, gp1, dis16, x, s16, W1, W2, bsum)
        embs.append(x)

    x1, x2, x3 = embs[1], embs[2], embs[3]
    user_emb = _asm_user(user_w, x1, x2, x3)
    item_emb = _asm_item(item_w, x1, x2, x3)
    return user_emb, item_emb


# speedup vs baseline: 2.3670x; 1.0753x over previous
import functools

import jax
import jax.numpy as jnp
from jax import lax
from jax.experimental import pallas as pl
from jax.experimental.pallas import tpu as pltpu
from jax.experimental.pallas import tpu_sc as plsc

N_USERS = 25000
N_ITEMS = 25000
N = N_USERS + N_ITEMS
E = 800000
D = 64
H = 32

NC = 2
NS = 16
CH = 128

N_PAD = 50176
STRIPE = N_PAD // NS
E_PAD = 819200
K = E_PAD // CH
NGC2 = K // (NC * NS)
NGC3 = K // NS
RING = 8
VR = 4

_mesh = plsc.VectorSubcoreMesh(core_axis_name="c", subcore_axis_name="s")
_sc_params = pltpu.CompilerParams(use_tc_tiling_on_sc=False)


def _out2(width):
    return [jax.ShapeDtypeStruct((N_PAD, width), jnp.float32),
            jax.ShapeDtypeStruct((N_PAD, width), jnp.float32)]


@functools.partial(
    pl.kernel,
    out_type=_out2(16),
    mesh=_mesh,
    compiler_params=_sc_params,
    scratch_types=[
        pltpu.VMEM((RING, CH), jnp.int32),
        pltpu.VMEM((CH, 16), jnp.float32),
        pltpu.VMEM_SHARED((N_PAD, 16), jnp.float32),
        pltpu.SemaphoreType.DMA,
        pltpu.SemaphoreType.DMA,
    ],
)
def _sc_deg(colk_hbm, ones_hbm, zeros_hbm, out0_hbm, out1_hbm,
            cidx, ones_v, accum, isem, ssem):
    c = lax.axis_index("c")
    s = lax.axis_index("s")
    base = (c * NS + s) * NGC2
    for j in range(4):
        pltpu.async_copy(colk_hbm.at[base + j], cidx.at[j], isem)
    pltpu.sync_copy(ones_hbm, ones_v)
    pltpu.sync_copy(zeros_hbm, accum.at[pl.ds(s * STRIPE, STRIPE)])
    plsc.subcore_barrier()

    @pl.loop(0, NGC2 // RING)
    def _(u):
        for j in range(RING):
            g = u * RING + j

            @pl.when(g >= 2)
            def _():
                pltpu.make_async_copy(ones_v, accum.at[cidx.at[j]], ssem).wait()

            @pl.when(g + 4 < NGC2)
            def _():
                pltpu.async_copy(colk_hbm.at[base + g + 4],
                                 cidx.at[(j + 4) % RING], isem)
            pltpu.make_async_copy(colk_hbm.at[base + g], cidx.at[j],
                                  isem).wait()
            pltpu.async_copy(ones_v, accum.at[cidx.at[j]], ssem, add=True)

    pltpu.make_async_copy(ones_v, accum.at[cidx.at[0]], ssem).wait()
    pltpu.make_async_copy(ones_v, accum.at[cidx.at[0]], ssem).wait()
    plsc.subcore_barrier()
    out = [out0_hbm, out1_hbm]
    for cc in range(NC):
        @pl.when(c == cc)
        def _():
            pltpu.sync_copy(accum.at[pl.ds(s * STRIPE, STRIPE)],
                            out[cc].at[pl.ds(s * STRIPE, STRIPE)])


def _pipelined_gather_scatter(table_hbm, rowk_hbm, colk_hbm, rbase, cbase,
                              ridx, cidx, vbufs, accum, isem, gsem, ssem,
                              ngc):
    for j in range(4):
        pltpu.async_copy(rowk_hbm.at[rbase + j], ridx.at[j], isem)
        pltpu.async_copy(colk_hbm.at[cbase + j], cidx.at[j], isem)
    for j in range(2):
        pltpu.make_async_copy(rowk_hbm.at[rbase], ridx.at[j], isem).wait()
        pltpu.make_async_copy(colk_hbm.at[cbase], cidx.at[j], isem).wait()
        pltpu.async_copy(table_hbm.at[ridx.at[j]], vbufs[j], gsem)

    @pl.loop(0, ngc // RING)
    def _(u):
        for j in range(RING):
            g = u * RING + j
            vb = vbufs[j % VR]

            @pl.when(g >= 2)
            def _():
                pltpu.make_async_copy(vb, accum.at[cidx.at[j]], ssem).wait()

            @pl.when(g + 4 < ngc)
            def _():
                jp = (j + 4) % RING
                pltpu.async_copy(rowk_hbm.at[rbase + g + 4], ridx.at[jp], isem)
                pltpu.async_copy(colk_hbm.at[cbase + g + 4], cidx.at[jp], isem)

            @pl.when(g + 2 < ngc)
            def _():
                jg = (j + 2) % RING
                pltpu.make_async_copy(rowk_hbm.at[rbase], ridx.at[jg],
                                      isem).wait()
                pltpu.make_async_copy(colk_hbm.at[cbase], cidx.at[jg],
                                      isem).wait()
                pltpu.async_copy(table_hbm.at[ridx.at[jg]],
                                 vbufs[(j + 2) % VR], gsem)
            pltpu.make_async_copy(table_hbm.at[ridx.at[j]], vb, gsem).wait()
            pltpu.async_copy(vb, accum.at[cidx.at[j]], ssem, add=True)

    pltpu.make_async_copy(vbufs[0], accum.at[cidx.at[0]], ssem).wait()
    pltpu.make_async_copy(vbufs[0], accum.at[cidx.at[0]], ssem).wait()


def _make_seg_kernel(width, ngc, two_core_scan, wide_out=False):
    out_w = 128 if wide_out else width
    @functools.partial(
        pl.kernel,
        out_type=[jax.ShapeDtypeStruct((N_PAD, out_w), jnp.float32),
                  jax.ShapeDtypeStruct((N_PAD, out_w), jnp.float32)],
        mesh=_mesh,
        compiler_params=_sc_params,
        scratch_types=[
            pltpu.VMEM((RING, CH), jnp.int32),
            pltpu.VMEM((RING, CH), jnp.int32),
            pltpu.VMEM((CH, width), jnp.float32),
            pltpu.VMEM((CH, width), jnp.float32),
            pltpu.VMEM((CH, width), jnp.float32),
            pltpu.VMEM((CH, width), jnp.float32),
            pltpu.VMEM_SHARED((N_PAD, width), jnp.float32),
            pltpu.SemaphoreType.DMA,
            pltpu.SemaphoreType.DMA,
            pltpu.SemaphoreType.DMA,
        ],
    )
    def seg(table_hbm, rowk_hbm, colk_hbm, zeros_hbm, out0_hbm, out1_hbm,
            ridx, cidx, v0, v1, v2, v3, accum, isem, gsem, ssem):
        c = lax.axis_index("c")
        s = lax.axis_index("s")
        if two_core_scan:
            rbase = c * K + s * ngc
            cbase = s * ngc
        else:
            rbase = (c * NS + s) * ngc
            cbase = rbase
        pltpu.sync_copy(zeros_hbm, accum.at[pl.ds(s * STRIPE, STRIPE)])
        plsc.subcore_barrier()
        _pipelined_gather_scatter(table_hbm, rowk_hbm, colk_hbm, rbase, cbase,
                                  ridx, cidx, (v0, v1, v2, v3),
                                  accum, isem, gsem, ssem, ngc)
        plsc.subcore_barrier()
        out = [out0_hbm, out1_hbm]
        for cc in range(NC):
            @pl.when(c == cc)
            def _():
                if wide_out:
                    pltpu.sync_copy(
                        accum.at[pl.ds(s * STRIPE, STRIPE)],
                        out[cc].at[pl.ds(s * STRIPE, STRIPE), pl.ds(0, width)])
                else:
                    pltpu.sync_copy(accum.at[pl.ds(s * STRIPE, STRIPE)],
                                    out[cc].at[pl.ds(s * STRIPE, STRIPE)])

    return seg


_sc_t = _make_seg_kernel(16, NGC2, False)
_sc_seg = _make_seg_kernel(H, NGC3, True, wide_out=True)


NB = 3584
PBD = NB * 16 // 128


def _tc_dis_body(p0_ref, p1_ref, dpack_ref):
    deg = p0_ref[...] + p1_ref[...]
    dpack_ref[...] = jnp.where(deg > 0, lax.rsqrt(deg), 0.0)


_tc_dis = pl.pallas_call(
    _tc_dis_body,
    grid=(N_PAD // NB,),
    in_specs=[pl.BlockSpec((PBD, 128), lambda i: (i, 0)),
              pl.BlockSpec((PBD, 128), lambda i: (i, 0))],
    out_specs=pl.BlockSpec((PBD, 128), lambda i: (i, 0)),
    out_shape=jax.ShapeDtypeStruct((N_PAD * 16 // 128, 128), jnp.float32),
)


def _pack_y(y, yp_ref):
    v = y.reshape(y.shape[0] // 2, 2, D)
    yp_ref[:, 0:D] = v[:, 0, :]
    yp_ref[:, D:2 * D] = v[:, 1, :]


def _tc_prep_body(p0_ref, p1_ref, d_ref, x0_ref, s16_ref, yp_ref):
    t = p0_ref[...] + p1_ref[...]
    d16 = d_ref[...]
    s16_ref[...] = d16 * t
    _pack_y(d16[:, 0:1] * x0_ref[...], yp_ref)


_tc_prep = pl.pallas_call(
    _tc_prep_body,
    grid=(N_PAD // NB,),
    in_specs=[
        pl.BlockSpec((NB, 16), lambda i: (i, 0)),
        pl.BlockSpec((NB, 16), lambda i: (i, 0)),
        pl.BlockSpec((NB, 16), lambda i: (i, 0)),
        pl.BlockSpec((NB, D), lambda i: (i, 0)),
    ],
    out_specs=[
        pl.BlockSpec((NB, 16), lambda i: (i, 0)),
        pl.BlockSpec((NB // 2, 128), lambda i: (i, 0)),
    ],
    out_shape=[
        jax.ShapeDtypeStruct((N_PAD, 16), jnp.float32),
        jax.ShapeDtypeStruct((N_PAD // 2, 128), jnp.float32),
    ],
)


def _tc_layer_body(g0_ref, g1_ref, d_ref, x_ref, s16_ref, w1_ref, w2_ref,
                   b_ref, xn_ref, y_ref):
    graw = jnp.concatenate([g0_ref[...][:, :H], g1_ref[...][:, :H]], axis=1)
    d1 = d_ref[...][:, 0:1]
    a = d1 * graw
    x = x_ref[...]
    acc = lax.dot_general(a, w1_ref[...], (((1,), (1,)), ((), ())),
                          preferred_element_type=jnp.float32)
    acc = acc + lax.dot_general(x * a, w2_ref[...], (((1,), (1,)), ((), ())),
                                preferred_element_type=jnp.float32)
    acc = acc + s16_ref[...][:, 0:1] * b_ref[...]
    xn = jnp.where(acc >= 0, acc, 0.01 * acc)
    xn_ref[...] = xn
    _pack_y(d1 * xn, y_ref)


_tc_layer = pl.pallas_call(
    _tc_layer_body,
    grid=(N_PAD // NB,),
    in_specs=[
        pl.BlockSpec((NB, 128), lambda i: (i, 0)),
        pl.BlockSpec((NB, 128), lambda i: (i, 0)),
        pl.BlockSpec((NB, 16), lambda i: (i, 0)),
        pl.BlockSpec((NB, D), lambda i: (i, 0)),
        pl.BlockSpec((NB, 16), lambda i: (i, 0)),
        pl.BlockSpec((D, D), lambda i: (0, 0)),
        pl.BlockSpec((D, D), lambda i: (0, 0)),
        pl.BlockSpec((1, D), lambda i: (0, 0)),
    ],
    out_specs=[
        pl.BlockSpec((NB, D), lambda i: (i, 0)),
        pl.BlockSpec((NB // 2, 128), lambda i: (i, 0)),
    ],
    out_shape=[
        jax.ShapeDtypeStruct((N_PAD, D), jnp.float32),
        jax.ShapeDtypeStruct((N_PAD // 2, 128), jnp.float32),
    ],
)


NBA = 5000


def _asm_body(w_ref, x1_ref, x2_ref, x3_ref, o_ref):
    o_ref[...] = jnp.concatenate(
        [w_ref[...], x1_ref[...], x2_ref[...], x3_ref[...]], axis=1)


def _make_asm(row_off):
    blocks = N_USERS // NBA
    return pl.pallas_call(
        _asm_body,
        grid=(blocks,),
        in_specs=[
            pl.BlockSpec((NBA, D), lambda i: (i, 0)),
            pl.BlockSpec((NBA, D), lambda i, o=row_off // NBA: (i + o, 0)),
            pl.BlockSpec((NBA, D), lambda i, o=row_off // NBA: (i + o, 0)),
            pl.BlockSpec((NBA, D), lambda i, o=row_off // NBA: (i + o, 0)),
        ],
        out_specs=pl.BlockSpec((NBA, 4 * D), lambda i: (i, 0)),
        out_shape=jax.ShapeDtypeStruct((N_USERS, 4 * D), jnp.float32),
    )


_asm_user = _make_asm(0)
_asm_item = _make_asm(N_USERS)


def _to_sc_table(y, rows, width):
    pack = lax.optimization_barrier(y.reshape(rows * width // 128, 128))
    return pack.reshape(rows, width)


def kernel(user_w, item_w, W1_0, b1_0, W2_0, b2_0, W1_1, b1_1, W2_1, b2_1,
           W1_2, b1_2, W2_2, b2_2, edge_index):
    row = edge_index[0].astype(jnp.int32)
    col = edge_index[1].astype(jnp.int32)
    pad = E_PAD - E
    pad_iota = lax.iota(jnp.int32, pad)
    colp = jnp.concatenate([col, N + pad_iota % (N_PAD - N)])
    rowp = jnp.concatenate([row, pad_iota % N])
    colk = colp.reshape(K, CH)
    rowk = rowp.reshape(K, CH)
    rowy = jnp.concatenate([2 * rowp, 2 * rowp + 1]).reshape(2 * K, CH)

    ones16 = jnp.ones((16, CH), jnp.float32).reshape(CH, 16)
    zeros16 = jnp.zeros((STRIPE * 16 // 128, 128), jnp.float32)
    zeros32 = jnp.zeros((STRIPE * H // 128, 128), jnp.float32)
    x0 = jnp.concatenate([user_w, item_w,
                          jnp.zeros((N_PAD - N, D), jnp.float32)], axis=0)

    dp0, dp1 = _sc_deg(colk, ones16, zeros16.reshape(STRIPE, 16))
    dpack = _tc_dis(dp0.reshape(N_PAD * 16 // 128, 128),
                    dp1.reshape(N_PAD * 16 // 128, 128))
    distab = dpack.reshape(N_PAD, 16)
    dis16 = dpack.reshape(N_PAD, 16)
    tp0, tp1 = _sc_t(distab, rowk, colk, zeros16.reshape(STRIPE, 16))
    s16, y = _tc_prep(tp0, tp1, dis16, x0)

    params = [(W1_0, b1_0, W2_0, b2_0), (W1_1, b1_1, W2_1, b2_1),
              (W1_2, b1_2, W2_2, b2_2)]
    embs = [x0]
    x = x0
    for (W1, b1, W2, b2) in params:
        ytab = y.reshape(2 * N_PAD, H)
        gp0, gp1 = _sc_seg(ytab, rowy, colk, zeros32.reshape(STRIPE, H))
        bsum = (b1 + b2).reshape(1, D)
        x, y = _tc_layer(gp0, gp1, dis16, x, s16, W1, W2, bsum)
        embs.append(x)

    x1, x2, x3 = embs[1], embs[2], embs[3]
    user_emb = _asm_user(user_w, x1, x2, x3)
    item_emb = _asm_item(item_w, x1, x2, x3)
    return user_emb, item_emb
